# Initial kernel scaffold; baseline (speedup 1.0000x reference)
import jax
import jax.numpy as jnp
from jax.experimental import pallas as pl


def kernel(x, neighbors, W1, b1, W2, b2, Wmu, bmu, Wdec, bdec, Wup, bup, Wfin, bfin):
    raise NotImplementedError("write your pallas kernel here")


# speedup vs baseline: 2.8298x; 2.8298x over previous
import functools

import jax
import jax.numpy as jnp
from jax import lax
from jax.experimental import pallas as pl
from jax.experimental.pallas import tpu as pltpu
from jax.experimental.pallas import tpu_sc as plsc

N = 50000
K = 6
C = 128
S = K + 1
EPS = 1e-5

NC = 2
NS = 16
NW = NC * NS

NPAD = 50176
RPT = NPAD // NW
SB = 112
NSB = RPT // SB
CH_OR = 16
CH_GR = CH_OR * S
CPS = SB // CH_OR

RB = 512
NRB = NPAD // RB


def _mm_body(mode, x_ref, *refs):
    if mode == "plain":
        (w_ref, y_ref) = refs
        t = x_ref[...]
    elif mode == "norm":
        (s_ref, w_ref, y_ref) = refs
        t = jnp.maximum((x_ref[...] - s_ref[0:1, :]) * s_ref[1:2, :], 0.0)
    elif mode == "norm_out":
        (s_ref, w_ref, y_ref, h_ref) = refs
        t = jnp.maximum((x_ref[...] - s_ref[0:1, :]) * s_ref[1:2, :], 0.0)
        h_ref[...] = t
    elif mode == "norm_res":
        (s_ref, r_ref, w_ref, y_ref) = refs
        t = jnp.maximum(
            (x_ref[...] - s_ref[0:1, :]) * s_ref[1:2, :] + r_ref[...], 0.0)
    y_ref[...] = jnp.dot(t, w_ref[...], preferred_element_type=jnp.float32)


def _mm(mode, x, stats, resid, wall, bias_row=None):
    cin = x.shape[1]
    cout7 = wall.shape[1]
    in_specs = [pl.BlockSpec((RB, cin), lambda i: (i, 0))]
    args = [x]
    if mode != "plain":
        in_specs.append(pl.BlockSpec((2, cin), lambda i: (0, 0)))
        args.append(stats)
    if mode == "norm_res":
        in_specs.append(pl.BlockSpec((RB, cin), lambda i: (i, 0)))
        args.append(resid)
    in_specs.append(pl.BlockSpec((cin, cout7), lambda i: (0, 0)))
    args.append(wall)
    out_shape = [jax.ShapeDtypeStruct((NPAD, cout7), jnp.float32)]
    out_specs = [pl.BlockSpec((RB, cout7), lambda i: (i, 0))]
    if mode == "norm_out":
        out_shape.append(jax.ShapeDtypeStruct((NPAD, cin), jnp.float32))
        out_specs.append(pl.BlockSpec((RB, cin), lambda i: (i, 0)))
    res = pl.pallas_call(
        functools.partial(_mm_body, mode),
        grid=(NRB,),
        in_specs=in_specs,
        out_specs=out_specs,
        out_shape=out_shape,
    )(*args)
    if bias_row is not None:
        raise NotImplementedError
    return res if mode == "norm_out" else res[0]


def _stats_body(x_ref, o_ref, acc_ref):
    i = pl.program_id(0)

    @pl.when(i == 0)
    def _():
        acc_ref[...] = jnp.zeros_like(acc_ref)

    row = lax.broadcasted_iota(jnp.int32, (RB, 1), 0) + i * RB
    t = jnp.where(row < N, x_ref[...], 0.0)
    acc_ref[0:1, :] += jnp.sum(t, axis=0, keepdims=True)
    acc_ref[1:2, :] += jnp.sum(t * t, axis=0, keepdims=True)

    @pl.when(i == NRB - 1)
    def _():
        m = acc_ref[0:1, :] * (1.0 / N)
        var = acc_ref[1:2, :] * (1.0 / N) - m * m
        o_ref[0:1, :] = m
        o_ref[1:2, :] = lax.rsqrt(var + EPS)


def _stats(p):
    return pl.pallas_call(
        _stats_body,
        grid=(NRB,),
        in_specs=[pl.BlockSpec((RB, C), lambda i: (i, 0))],
        out_specs=pl.BlockSpec((2, C), lambda i: (0, 0)),
        out_shape=jax.ShapeDtypeStruct((2, C), jnp.float32),
        scratch_shapes=[pltpu.VMEM((2, C), jnp.float32)],
    )(p)


def _final_body(x_ref, s_ref, w_ref, b_ref, o_ref):
    t = jnp.maximum((x_ref[...] - s_ref[0:1, :]) * s_ref[1:2, :], 0.0)
    y = jnp.dot(t, w_ref[...], preferred_element_type=jnp.float32)
    o_ref[...] = y + b_ref[...]


def _final_mm(p, stats, wall, bias_row):
    cout7 = wall.shape[1]
    return pl.pallas_call(
        _final_body,
        grid=(NRB,),
        in_specs=[
            pl.BlockSpec((RB, C), lambda i: (i, 0)),
            pl.BlockSpec((2, C), lambda i: (0, 0)),
            pl.BlockSpec((C, cout7), lambda i: (0, 0)),
            pl.BlockSpec((1, cout7), lambda i: (0, 0)),
        ],
        out_specs=pl.BlockSpec((RB, cout7), lambda i: (i, 0)),
        out_shape=jax.ShapeDtypeStruct((NPAD, cout7), jnp.float32),
    )(p, stats, wall, bias_row)


def _tanh_body(x_ref, o_ref):
    o_ref[...] = jnp.tanh(x_ref[...])


def _tanh(p4):
    co = p4.shape[1]
    return pl.pallas_call(
        _tanh_body,
        grid=(NRB,),
        in_specs=[pl.BlockSpec((RB, co), lambda i: (i, 0))],
        out_specs=pl.BlockSpec((RB, co), lambda i: (i, 0)),
        out_shape=jax.ShapeDtypeStruct((NPAD, co), jnp.float32),
    )(p4)


@functools.lru_cache(maxsize=None)
def _make_sc_gather(cout, packed=False):
    mesh = plsc.VectorSubcoreMesh(
        core_axis_name="c", subcore_axis_name="s",
        num_cores=NC, num_subcores=NS)

    gbytes = CH_GR * cout * 4
    abytes = SB * cout * 4

    @functools.partial(
        pl.kernel,
        out_type=jax.ShapeDtypeStruct((NPAD, cout), jnp.float32),
        mesh=mesh,
        scratch_types=[
            pltpu.VMEM((CPS * CH_GR,), jnp.int32),
            pltpu.VMEM((2, CH_GR, cout), jnp.float32),
            pltpu.VMEM((2, SB, cout), jnp.float32),
            pltpu.SemaphoreType.DMA,
            pltpu.SemaphoreType.DMA,
        ],
    )
    def sc_gather(idx_hbm, y_hbm, out_hbm, idxv, gbuf, accb, gsem, ssem):
        wid = lax.axis_index("s") * NC + lax.axis_index("c")
        row_base = wid * RPT

        def issue_gather(j, slot):
            pltpu.async_copy(
                y_hbm.at[idxv.at[pl.ds(j * CH_GR, CH_GR)]],
                gbuf.at[slot], gsem)

        def wait_gather(slot):
            pltpu.make_async_copy(
                y_hbm.at[pl.ds(0, CH_GR)], gbuf.at[slot], gsem).wait()

        def wait_store(ab):
            pltpu.make_async_copy(
                out_hbm.at[pl.ds(0, SB)], accb.at[ab], ssem).wait()

        def do_superblock(it, half):
            sb = it * 2 + half
            ab = half
            pltpu.sync_copy(
                idx_hbm.at[pl.ds((row_base + sb * SB) * S, CPS * CH_GR)],
                idxv)
            issue_gather(0, 0)
            @pl.when(sb >= 2)
            def _():
                wait_store(ab)
            for j in range(CPS):
                slot = j & 1
                if j + 1 < CPS:
                    issue_gather(j + 1, 1 - slot)
                wait_gather(slot)

                if packed:
                    @pl.loop(0, CH_OR)
                    def _(r):
                        v = gbuf[slot, r * S, pl.ds(0, 16)]
                        for t in range(1, S):
                            v = v + gbuf[slot, r * S + t, pl.ds(t * 16, 16)]
                        accb[ab, j * CH_OR + r, pl.ds(0, 16)] = v
                else:
                    @pl.loop(0, CH_OR)
                    def _(r):
                        for c in range(cout // 16):
                            cs = pl.ds(c * 16, 16)
                            v = gbuf[slot, r * S, cs]
                            for t in range(1, S):
                                v = v + gbuf[slot, r * S + t, cs]
                            accb[ab, j * CH_OR + r, cs] = v
            pltpu.async_copy(
                accb.at[ab],
                out_hbm.at[pl.ds(wid * RPT + sb * SB, SB)],
                ssem)

        @pl.loop(0, NSB // 2)
        def _(it):
            do_superblock(it, 0)
            do_superblock(it, 1)

        wait_store(0)
        wait_store(1)

    return sc_gather


def _sc_gather(idx_flat, yflat, cout, packed=False):
    return _make_sc_gather(cout, packed)(idx_flat, yflat)


def _wall(w):
    return jnp.transpose(w, (1, 0, 2)).reshape(w.shape[1], S * w.shape[2])


@jax.jit
def _run(x, neighbors, W1, W2, WupB, Wfin_p, bias_row):
    xp = jnp.pad(x, ((0, NPAD - N), (0, 0)))
    nb = jnp.pad(neighbors.astype(jnp.int32), ((0, NPAD - N), (0, 0)))
    self_idx = (jnp.arange(NPAD, dtype=jnp.int32) * S)[:, None]
    nbr_idx = nb * S + jnp.arange(1, S, dtype=jnp.int32)[None, :]
    idx = jnp.concatenate([self_idx, nbr_idx], axis=1)
    idx3 = idx.reshape(NPAD * S)
    idxn = jnp.concatenate(
        [jnp.arange(NPAD, dtype=jnp.int32)[:, None], nb], axis=1
    ).reshape(NPAD * S)

    wall1 = _wall(W1)
    wall2 = _wall(W2)
    wall3 = _wall(WupB)

    y1 = _mm("plain", xp, None, None, wall1)
    p1 = _sc_gather(idx3, y1.reshape(NPAD * S, C), C)
    s1 = _stats(p1)

    y2, h1 = _mm("norm_out", p1, s1, None, wall2)
    p2 = _sc_gather(idx3, y2.reshape(NPAD * S, C), C)
    s2 = _stats(p2)

    y3 = _mm("norm_res", p2, s2, h1, wall3)
    p3 = _sc_gather(idx3, y3.reshape(NPAD * S, C), C)
    s3 = _stats(p3)

    y4 = _final_mm(p3, s3, Wfin_p, bias_row)
    p4 = _sc_gather(idxn, y4, C, packed=True)
    o = _tanh(p4)
    return o[:N, :3]


def kernel(x, neighbors, W1, b1, W2, b2, Wmu, bmu, Wdec, bdec, Wup, bup,
           Wfin, bfin):
    WupB = Wup[:, C:, :]
    wfin_p = jnp.pad(Wfin, ((0, 0), (0, 0), (0, 16 - Wfin.shape[2])))
    wall4 = jnp.pad(_wall(wfin_p), ((0, 0), (0, C - S * 16)))
    bias_row = jnp.zeros((1, C), jnp.float32)
    bias_row = bias_row.at[0, :3].set(bfin)
    return _run(x, neighbors, W1, W2, WupB, wall4, bias_row)
